# baseline (speedup 1.0000x reference)
import jax
import jax.numpy as jnp
from jax import lax
from jax.experimental import pallas as pl
from jax.experimental.pallas import tpu as pltpu

EMBED_DIM = 16
DELTA_V = 0.5
DELTA_D = 3.0

_CHUNK = 3584


def _safe_norm(sq):
    pos = sq > 0
    safe = jnp.where(pos, sq, 1.0)
    return jnp.sqrt(safe) * pos.astype(sq.dtype)


def _loss_kernel(e_ref, m_ref, var_ref, dist_ref,
                 e_sv, m_sv, sums_sv, counts_sv, *, L, B, C):
    b = pl.program_id(0)
    p = pl.program_id(1)
    c = pl.program_id(2)

    @pl.when((b == 0) & (p == 0) & (c == 0))
    def _():
        var_ref[:, :] = jnp.zeros((1, 1), jnp.float32)
        dist_ref[:, :] = jnp.zeros((1, 1), jnp.float32)

    @pl.when(p == 0)
    def _phase0():
        e = e_ref[0]
        mf = m_ref[0].astype(jnp.float32)
        e_sv[:, pl.ds(c * _CHUNK, _CHUNK)] = e
        m_sv[:, pl.ds(c * _CHUNK, _CHUNK)] = mf
        part_sums = lax.dot_general(
            mf, e, (((1,), (1,)), ((), ())),
            preferred_element_type=jnp.float32)
        part_counts = jnp.sum(mf, axis=1, keepdims=True)

        @pl.when(c == 0)
        def _():
            sums_sv[:, :] = part_sums
            counts_sv[:, :] = part_counts

        @pl.when(c > 0)
        def _():
            sums_sv[:, :] += part_sums
            counts_sv[:, :] += part_counts

    @pl.when(p == 1)
    def _phase1():
        counts = counts_sv[:, :]
        means = sums_sv[:, :] / counts
        e = e_sv[:, pl.ds(c * _CHUNK, _CHUNK)]
        mf = m_sv[:, pl.ds(c * _CHUNK, _CHUNK)]

        enorm2 = jnp.sum(e * e, axis=0, keepdims=True)
        mnorm2 = jnp.sum(means * means, axis=1, keepdims=True)
        dot2 = lax.dot_general(
            means, e, (((1,), (0,)), ((), ())),
            preferred_element_type=jnp.float32)

        sq = jnp.maximum(enorm2 - 2.0 * dot2 + mnorm2, 0.0)
        norm = _safe_norm(sq)
        var_t = jnp.maximum(norm - DELTA_V, 0.0) ** 2 * mf
        lane_sums = jnp.sum(var_t, axis=1, keepdims=True) / counts
        var_ref[:, :] += (jnp.sum(lane_sums) / (L * B)).reshape(1, 1)

        @pl.when(c == 0)
        def _():
            cdiff = means[:, None, :] - means[None, :, :]
            dsq = jnp.sum(cdiff * cdiff, axis=2)
            eye = (lax.broadcasted_iota(jnp.int32, (L, L), 0)
                   == lax.broadcasted_iota(jnp.int32, (L, L), 1)
                   ).astype(jnp.float32)
            dist = _safe_norm(dsq) + eye * DELTA_D
            dist_terms = jnp.maximum(DELTA_D - dist, 0.0) ** 2
            dist_ref[:, :] += (
                jnp.sum(dist_terms) / (L * (L - 1)) / 2.0 / B).reshape(1, 1)


def kernel(embedding, seg_gt):
    B, D, H, W = embedding.shape
    L = seg_gt.shape[1]
    N = H * W
    C = N // _CHUNK

    e = embedding.reshape(B, D, N)
    m = seg_gt.reshape(B, L, N)

    var_loss, dist_loss = pl.pallas_call(
        lambda e_ref, m_ref, v_ref, d_ref, *scratch: _loss_kernel(
            e_ref, m_ref, v_ref, d_ref, *scratch, L=L, B=B, C=C),
        grid=(B, 2, C),
        in_specs=[
            pl.BlockSpec((1, D, _CHUNK),
                         lambda b, p, c: (b, 0, c * (1 - p) + (C - 1) * p)),
            pl.BlockSpec((1, L, _CHUNK),
                         lambda b, p, c: (b, 0, c * (1 - p) + (C - 1) * p)),
        ],
        out_specs=[
            pl.BlockSpec((1, 1), lambda b, p, c: (0, 0)),
            pl.BlockSpec((1, 1), lambda b, p, c: (0, 0)),
        ],
        out_shape=[
            jax.ShapeDtypeStruct((1, 1), jnp.float32),
            jax.ShapeDtypeStruct((1, 1), jnp.float32),
        ],
        scratch_shapes=[
            pltpu.VMEM((D, N), jnp.float32),
            pltpu.VMEM((L, N), jnp.float32),
            pltpu.VMEM((L, D), jnp.float32),
            pltpu.VMEM((L, 1), jnp.float32),
        ],
    )(e, m)

    reg_loss = jnp.zeros((), dtype=embedding.dtype)
    return (var_loss[0, 0], dist_loss[0, 0], reg_loss)


# speedup vs baseline: 2.3243x; 2.3243x over previous
import jax
import jax.numpy as jnp
from jax import lax
from jax.experimental import pallas as pl
from jax.experimental.pallas import tpu as pltpu

EMBED_DIM = 16
DELTA_V = 0.5
DELTA_D = 3.0

_S = 4


def _loss_kernel(e_hbm, m_hbm, var_ref, dist_ref, e_sv, m_sv, sems,
                 *, L, B, D, N):
    ns = N // _S
    copies = []
    k = 0
    for bi in range(B):
        per_b = []
        for s in range(_S):
            ce = pltpu.make_async_copy(
                e_hbm.at[bi, :, pl.ds(s * ns, ns)],
                e_sv.at[bi, :, pl.ds(s * ns, ns)], sems.at[k])
            cm = pltpu.make_async_copy(
                m_hbm.at[bi, :, pl.ds(s * ns, ns)],
                m_sv.at[bi, :, pl.ds(s * ns, ns)], sems.at[k + 1])
            per_b.append((ce, cm))
            k += 2
        copies.append(per_b)
    for per_b in copies:
        for ce, cm in per_b:
            ce.start()
            cm.start()

    var_total = jnp.zeros((), jnp.float32)
    dist_total = jnp.zeros((), jnp.float32)
    for bi in range(B):
        counts = jnp.zeros((L, 1), jnp.float32)
        sums = jnp.zeros((L, D), jnp.float32)
        for s in range(_S):
            ce, cm = copies[bi][s]
            ce.wait()
            cm.wait()
            e = e_sv[bi, :, pl.ds(s * ns, ns)]
            mf = m_sv[bi, :, pl.ds(s * ns, ns)].astype(jnp.float32)
            counts += jnp.sum(mf, axis=1, keepdims=True)
            sums += lax.dot_general(
                mf, e, (((1,), (1,)), ((), ())),
                preferred_element_type=jnp.float32)
        means = sums / counts
        mnorm2 = jnp.sum(means * means, axis=1, keepdims=True)

        var_num = jnp.zeros((L, 1), jnp.float32)
        for s in range(_S):
            e = e_sv[bi, :, pl.ds(s * ns, ns)]
            mf = m_sv[bi, :, pl.ds(s * ns, ns)].astype(jnp.float32)
            enorm2 = jnp.sum(e * e, axis=0, keepdims=True)
            dot2 = lax.dot_general(
                means, e, (((1,), (0,)), ((), ())),
                preferred_element_type=jnp.float32)
            sq = jnp.maximum((enorm2 + mnorm2) - 2.0 * dot2, 0.0)
            norm = jnp.sqrt(sq)
            var_t = jnp.maximum(norm - DELTA_V, 0.0) ** 2 * mf
            var_num += jnp.sum(var_t, axis=1, keepdims=True)
        var_total += jnp.sum(var_num / counts) / (L * B)

        cdiff = means[:, None, :] - means[None, :, :]
        dsq = jnp.sum(cdiff * cdiff, axis=2)
        eye = (lax.broadcasted_iota(jnp.int32, (L, L), 0)
               == lax.broadcasted_iota(jnp.int32, (L, L), 1)
               ).astype(jnp.float32)
        dist = jnp.sqrt(jnp.maximum(dsq, 0.0)) + eye * DELTA_D
        dist_terms = jnp.maximum(DELTA_D - dist, 0.0) ** 2
        dist_total += jnp.sum(dist_terms) / (L * (L - 1)) / 2.0 / B

    var_ref[:, :] = var_total.reshape(1, 1)
    dist_ref[:, :] = dist_total.reshape(1, 1)


def kernel(embedding, seg_gt):
    B, D, H, W = embedding.shape
    L = seg_gt.shape[1]
    N = H * W

    e = embedding.reshape(B, D, N)
    m = seg_gt.reshape(B, L, N)

    var_loss, dist_loss = pl.pallas_call(
        lambda e_ref, m_ref, v_ref, d_ref, *scratch: _loss_kernel(
            e_ref, m_ref, v_ref, d_ref, *scratch, L=L, B=B, D=D, N=N),
        in_specs=[
            pl.BlockSpec(memory_space=pltpu.MemorySpace.HBM),
            pl.BlockSpec(memory_space=pltpu.MemorySpace.HBM),
        ],
        out_specs=[
            pl.BlockSpec(memory_space=pltpu.MemorySpace.VMEM),
            pl.BlockSpec(memory_space=pltpu.MemorySpace.VMEM),
        ],
        out_shape=[
            jax.ShapeDtypeStruct((1, 1), jnp.float32),
            jax.ShapeDtypeStruct((1, 1), jnp.float32),
        ],
        scratch_shapes=[
            pltpu.VMEM((B, D, N), jnp.float32),
            pltpu.VMEM((B, L, N), m.dtype),
            pltpu.SemaphoreType.DMA((2 * _S * B,)),
        ],
    )(e, m)

    reg_loss = jnp.zeros((), dtype=embedding.dtype)
    return (var_loss[0, 0], dist_loss[0, 0], reg_loss)


# speedup vs baseline: 2.4483x; 1.0533x over previous
import jax
import jax.numpy as jnp
from jax import lax
from jax.experimental import pallas as pl
from jax.experimental.pallas import tpu as pltpu

EMBED_DIM = 16
DELTA_V = 0.5
DELTA_D = 3.0

_S = 4


def _loss_kernel(e_hbm, m_hbm, var_ref, dist_ref, e_sv, m_sv, en_sv, sems,
                 *, L, B, D, N):
    ns = N // _S
    copies = []
    k = 0
    for bi in range(B):
        per_b = []
        for s in range(_S):
            ce = pltpu.make_async_copy(
                e_hbm.at[bi, :, pl.ds(s * ns, ns)],
                e_sv.at[bi, :, pl.ds(s * ns, ns)], sems.at[k])
            cm = pltpu.make_async_copy(
                m_hbm.at[bi, :, pl.ds(s * ns, ns)],
                m_sv.at[bi, :, pl.ds(s * ns, ns)], sems.at[k + 1])
            per_b.append((ce, cm))
            k += 2
        copies.append(per_b)
    for per_b in copies:
        for ce, cm in per_b:
            ce.start()
            cm.start()

    var_total = jnp.zeros((), jnp.float32)
    dist_total = jnp.zeros((), jnp.float32)
    ones_d = jnp.ones((1, D), jnp.float32)
    for bi in range(B):
        counts = jnp.zeros((L, 1), jnp.float32)
        sums = jnp.zeros((L, D), jnp.float32)
        for s in range(_S):
            ce, cm = copies[bi][s]
            ce.wait()
            cm.wait()
            e = e_sv[bi, :, pl.ds(s * ns, ns)]
            mf = m_sv[bi, :, pl.ds(s * ns, ns)].astype(jnp.float32)
            counts += jnp.sum(mf, axis=1, keepdims=True)
            sums += lax.dot_general(
                mf, e, (((1,), (1,)), ((), ())),
                preferred_element_type=jnp.float32)
            en_sv[bi, :, pl.ds(s * ns, ns)] = lax.dot_general(
                ones_d, e * e, (((1,), (0,)), ((), ())),
                preferred_element_type=jnp.float32)
        means = sums / counts
        mnorm2 = jnp.sum(means * means, axis=1, keepdims=True)

        var_num = jnp.zeros((L, 1), jnp.float32)
        for s in range(_S):
            e = e_sv[bi, :, pl.ds(s * ns, ns)]
            mi = m_sv[bi, :, pl.ds(s * ns, ns)]
            enorm2 = en_sv[bi, :, pl.ds(s * ns, ns)]
            dot2 = lax.dot_general(
                means, e, (((1,), (0,)), ((), ())),
                preferred_element_type=jnp.float32)
            sq = jnp.maximum((enorm2 + mnorm2) - 2.0 * dot2, 0.0)
            norm = sq * lax.rsqrt(sq + 1e-20)
            var_t = jnp.where(mi > 0,
                              jnp.maximum(norm - DELTA_V, 0.0) ** 2, 0.0)
            var_num += jnp.sum(var_t, axis=1, keepdims=True)
        var_total += jnp.sum(var_num / counts) / (L * B)

        cdiff = means[:, None, :] - means[None, :, :]
        dsq = jnp.sum(cdiff * cdiff, axis=2)
        eye = (lax.broadcasted_iota(jnp.int32, (L, L), 0)
               == lax.broadcasted_iota(jnp.int32, (L, L), 1)
               ).astype(jnp.float32)
        dist = jnp.sqrt(jnp.maximum(dsq, 0.0)) + eye * DELTA_D
        dist_terms = jnp.maximum(DELTA_D - dist, 0.0) ** 2
        dist_total += jnp.sum(dist_terms) / (L * (L - 1)) / 2.0 / B

    var_ref[:, :] = var_total.reshape(1, 1)
    dist_ref[:, :] = dist_total.reshape(1, 1)


def kernel(embedding, seg_gt):
    B, D, H, W = embedding.shape
    L = seg_gt.shape[1]
    N = H * W

    e = embedding.reshape(B, D, N)
    m = seg_gt.reshape(B, L, N)

    var_loss, dist_loss = pl.pallas_call(
        lambda e_ref, m_ref, v_ref, d_ref, *scratch: _loss_kernel(
            e_ref, m_ref, v_ref, d_ref, *scratch, L=L, B=B, D=D, N=N),
        in_specs=[
            pl.BlockSpec(memory_space=pltpu.MemorySpace.HBM),
            pl.BlockSpec(memory_space=pltpu.MemorySpace.HBM),
        ],
        out_specs=[
            pl.BlockSpec(memory_space=pltpu.MemorySpace.VMEM),
            pl.BlockSpec(memory_space=pltpu.MemorySpace.VMEM),
        ],
        out_shape=[
            jax.ShapeDtypeStruct((1, 1), jnp.float32),
            jax.ShapeDtypeStruct((1, 1), jnp.float32),
        ],
        scratch_shapes=[
            pltpu.VMEM((B, D, N), jnp.float32),
            pltpu.VMEM((B, L, N), m.dtype),
            pltpu.VMEM((B, 1, N), jnp.float32),
            pltpu.SemaphoreType.DMA((2 * _S * B,)),
        ],
    )(e, m)

    reg_loss = jnp.zeros((), dtype=embedding.dtype)
    return (var_loss[0, 0], dist_loss[0, 0], reg_loss)
